# Initial kernel scaffold; baseline (speedup 1.0000x reference)
import jax
import jax.numpy as jnp
from jax.experimental import pallas as pl


def kernel(x, edge_index, W1, b1, W2, b2, Wc, bc):
    raise NotImplementedError("write your pallas kernel here")


# speedup vs baseline: 3.5029x; 3.5029x over previous
import functools

import jax
import jax.numpy as jnp
from jax import lax
from jax.experimental import pallas as pl
from jax.experimental.pallas import tpu as pltpu
from jax.experimental.pallas import tpu_sc as plsc

N = 10000
E = 160000
IN_DIM = 256
HIDDEN = 512
OUT_DIM = 16

f32 = jnp.float32
i32 = jnp.int32

NCORE = 2
NSUB = 16
CW = 128
NB1 = IN_DIM // CW
NB2 = HIDDEN // CW
CHUNK = 128
EP = E // NSUB
NCH = 80
EPP = NCH * CHUNK

NRANGE = 3
RSPAN = 3456
TBL_ROWS = RSPAN + CHUNK
TSTRIPE = TBL_ROWS // NSUB
OR_FULL = RSPAN // NSUB
RLAST_ROWS = N - (NRANGE - 1) * RSPAN
OR_LAST = 192
OR_LAST_TAIL = RLAST_ROWS - (NSUB - 1) * OR_LAST

_mesh = plsc.VectorSubcoreMesh(core_axis_name="c", subcore_axis_name="s")


def _fill(buf, rows, width, value):
    lanes = width // 16

    def body(i, carry):
        r = i // lanes
        col = (i % lanes) * 16
        buf[r, pl.ds(col, 16)] = jnp.full((16,), value, f32)
        return carry

    lax.fori_loop(0, rows * lanes, body, 0)


def _range_writeout(table, out, s, r):
    if r < NRANGE - 1:
        loc = pl.multiple_of(s * OR_FULL, 8)
        pltpu.sync_copy(table.at[pl.ds(loc, OR_FULL)],
                        out.at[pl.ds(pl.multiple_of(r * RSPAN + s * OR_FULL, 8),
                                     OR_FULL)])
    else:
        base = (NRANGE - 1) * RSPAN

        @pl.when(s < NSUB - 1)
        def _():
            loc = pl.multiple_of(s * OR_LAST, 8)
            pltpu.sync_copy(table.at[pl.ds(loc, OR_LAST)],
                            out.at[pl.ds(pl.multiple_of(base + s * OR_LAST, 8),
                                         OR_LAST)])

        @pl.when(s == NSUB - 1)
        def _():
            loc = (NSUB - 1) * OR_LAST
            pltpu.sync_copy(table.at[pl.ds(loc, OR_LAST_TAIL)],
                            out.at[pl.ds(base + loc, OR_LAST_TAIL)])


@functools.partial(
    pl.kernel,
    out_type=jax.ShapeDtypeStruct((N, 16), f32),
    mesh=_mesh,
    scratch_types=[pltpu.VMEM((NCH, CHUNK), i32)] * NRANGE + [
        pltpu.VMEM((CHUNK, 16), f32),
        pltpu.VMEM((TSTRIPE, 16), f32),
        pltpu.VMEM_SHARED((TBL_ROWS, 16), f32),
    ],
)
def _deg_kernel(*refs):
    dst_hbms = refs[:NRANGE]
    deg_hbm = refs[NRANGE]
    dst_vs = refs[NRANGE + 1:2 * NRANGE + 1]
    ones_v, init_v, table = refs[2 * NRANGE + 1:]
    c = lax.axis_index("c")
    s = lax.axis_index("s")
    for dh, dv in zip(dst_hbms, dst_vs):
        pltpu.sync_copy(dh.at[s], dv)
    _fill(ones_v, CHUNK, 16, 1.0)
    _fill(init_v, TSTRIPE, 16, 1.0)
    for r, dst_v in enumerate(dst_vs):
        pltpu.sync_copy(init_v,
                        table.at[pl.ds(pl.multiple_of(s * TSTRIPE, 8), TSTRIPE)])
        plsc.subcore_barrier()

        def body(j, carry, dst_v=dst_v):
            pltpu.sync_copy(ones_v, table.at[dst_v.at[j]], add=True)
            return carry

        lax.fori_loop(0, NCH, body, 0)
        plsc.subcore_barrier()

        @pl.when(c == 0)
        def _(r=r):
            _range_writeout(table, deg_hbm, s, r)

        plsc.subcore_barrier()


def _prop_one_phase(tbl, out, src_v, dst_v, buf0, buf1, zbuf, table,
                    sem0, sem1, s, r):
    pltpu.sync_copy(zbuf,
                    table.at[pl.ds(pl.multiple_of(s * TSTRIPE, 8), TSTRIPE)])
    plsc.subcore_barrier()

    pltpu.async_copy(tbl.at[src_v.at[0]], buf0, sem0)

    def body(i, carry):
        j0 = 2 * i
        pltpu.make_async_copy(tbl.at[src_v.at[j0]], buf0, sem0).wait()
        pltpu.async_copy(tbl.at[src_v.at[j0 + 1]], buf1, sem1)
        pltpu.sync_copy(buf0, table.at[dst_v.at[j0]], add=True)
        pltpu.make_async_copy(tbl.at[src_v.at[j0 + 1]], buf1, sem1).wait()

        @pl.when(j0 + 2 < NCH)
        def _():
            pltpu.async_copy(tbl.at[src_v.at[j0 + 2]], buf0, sem0)

        pltpu.sync_copy(buf1, table.at[dst_v.at[j0 + 1]], add=True)
        return carry

    lax.fori_loop(0, NCH // 2, body, 0)
    plsc.subcore_barrier()
    _range_writeout(table, out, s, r)
    plsc.subcore_barrier()


def _make_prop(nb):
    @functools.partial(
        pl.kernel,
        out_type=[jax.ShapeDtypeStruct((N, CW), f32) for _ in range(nb)],
        mesh=_mesh,
        scratch_types=[pltpu.VMEM((NCH, CHUNK), i32)] * (NRANGE + 1) + [
            pltpu.VMEM((CHUNK, CW), f32),
            pltpu.VMEM((CHUNK, CW), f32),
            pltpu.VMEM((TSTRIPE, CW), f32),
            pltpu.VMEM_SHARED((TBL_ROWS, CW), f32),
            pltpu.SemaphoreType.DMA,
            pltpu.SemaphoreType.DMA,
        ],
    )
    def _prop(src_hbm, *rest):
        dst_hbms = rest[:NRANGE]
        tbls = rest[NRANGE:NRANGE + nb]
        outs = rest[NRANGE + nb:NRANGE + 2 * nb]
        rest = rest[NRANGE + 2 * nb:]
        src_v = rest[0]
        dst_vs = rest[1:1 + NRANGE]
        buf0, buf1, zbuf, table, sem0, sem1 = rest[1 + NRANGE:]
        c = lax.axis_index("c")
        s = lax.axis_index("s")
        pltpu.sync_copy(src_hbm.at[s], src_v)
        for dh, dv in zip(dst_hbms, dst_vs):
            pltpu.sync_copy(dh.at[s], dv)
        _fill(zbuf, TSTRIPE, CW, 0.0)
        for cc in range(NCORE):
            @pl.when(c == cc)
            def _(cc=cc):
                for cb in range(cc, nb, NCORE):
                    for r, dst_v in enumerate(dst_vs):
                        _prop_one_phase(tbls[cb], outs[cb], src_v, dst_v,
                                        buf0, buf1, zbuf, table, sem0, sem1,
                                        s, r)

    return _prop


_prop_l1 = _make_prop(NB1)
_prop_l2 = _make_prop(NB2)


RB = 1000
GRID = N // RB


def _row_spec(width):
    return pl.BlockSpec((RB, width), lambda i: (i, 0))


def _full_spec(r, c):
    return pl.BlockSpec((r, c), lambda i: (0, 0))


def _tc_a_body(deg_ref, x_ref, dis_ref, *outs):
    d = lax.rsqrt(deg_ref[...])
    dis_ref[...] = d
    xs = x_ref[...] * d[:, 0:1]
    for k in range(NB1):
        outs[k][...] = xs[:, CW * k:CW * (k + 1)]


_tc_a = pl.pallas_call(
    _tc_a_body,
    grid=(GRID,),
    in_specs=[_row_spec(16), _row_spec(IN_DIM)],
    out_specs=[_row_spec(16)] + [_row_spec(CW)] * NB1,
    out_shape=[jax.ShapeDtypeStruct((N, 16), f32)]
    + [jax.ShapeDtypeStruct((N, CW), f32) for _ in range(NB1)],
)


def _tc_b_body(*refs):
    gs = refs[:NB1]
    x_ref, dis_ref, w1_ref, b1_ref = refs[NB1:NB1 + 4]
    outs = refs[NB1 + 4:]
    d = dis_ref[:, 0:1]
    g = jnp.concatenate([r[...] for r in gs], axis=1)
    p = d * (g + d * x_ref[...])
    h = jnp.dot(p, w1_ref[...], preferred_element_type=f32) + b1_ref[...]
    hs = d * jnp.maximum(h, 0.0)
    for k in range(NB2):
        outs[k][...] = hs[:, CW * k:CW * (k + 1)]


_tc_b = pl.pallas_call(
    _tc_b_body,
    grid=(GRID,),
    in_specs=[_row_spec(CW)] * NB1 + [_row_spec(IN_DIM), _row_spec(16),
              _full_spec(IN_DIM, HIDDEN), _full_spec(1, HIDDEN)],
    out_specs=[_row_spec(CW)] * NB2,
    out_shape=[jax.ShapeDtypeStruct((N, CW), f32) for _ in range(NB2)],
)


def _tc_c_body(*refs):
    gs = refs[:NB2]
    hs_refs = refs[NB2:2 * NB2]
    dis_ref, w2_ref, b2_ref, wc_ref, bc_ref, out_ref = refs[2 * NB2:]
    d = dis_ref[:, 0:1]
    g = jnp.concatenate([r[...] for r in gs], axis=1)
    hs = jnp.concatenate([r[...] for r in hs_refs], axis=1)
    p = d * (g + hs)
    h = jnp.dot(p, w2_ref[...], preferred_element_type=f32) + b2_ref[...]
    h = jnp.maximum(h, 0.0)
    out_ref[...] = jnp.dot(h, wc_ref[...], preferred_element_type=f32) + bc_ref[...]


_tc_c = pl.pallas_call(
    _tc_c_body,
    grid=(GRID,),
    in_specs=[_row_spec(CW)] * (2 * NB2) + [_row_spec(16),
              _full_spec(HIDDEN, HIDDEN), _full_spec(1, HIDDEN),
              _full_spec(HIDDEN, OUT_DIM), _full_spec(1, OUT_DIM)],
    out_specs=_row_spec(OUT_DIM),
    out_shape=jax.ShapeDtypeStruct((N, OUT_DIM), f32),
)


def kernel(x, edge_index, W1, b1, W2, b2, Wc, bc):
    src = edge_index[0].astype(i32)
    dst = edge_index[1].astype(i32)
    pad = EPP - EP
    srcr = jnp.concatenate(
        [src.reshape(NSUB, EP), jnp.zeros((NSUB, pad), i32)],
        axis=1).reshape(NSUB, NCH, CHUNK)
    dstr = jnp.concatenate(
        [dst.reshape(NSUB, EP), jnp.full((NSUB, pad), N, i32)],
        axis=1).reshape(NSUB, NCH, CHUNK)
    lane = jnp.arange(CHUNK, dtype=i32)[None, None, :]
    dsts = []
    for r in range(NRANGE):
        lo = r * RSPAN
        hi = min((r + 1) * RSPAN, N)
        inr = (dstr >= lo) & (dstr < hi)
        dsts.append(jnp.where(inr, dstr - lo, RSPAN + lane))

    deg16 = _deg_kernel(*dsts)
    dis, *xs = _tc_a(deg16, x)
    g1 = _prop_l1(srcr, *dsts, *xs)
    hs = _tc_b(*g1, x, dis, W1, b1.reshape(1, HIDDEN))
    g2 = _prop_l2(srcr, *dsts, *hs)
    out = _tc_c(*g2, *hs, dis, W2, b2.reshape(1, HIDDEN),
                Wc, bc.reshape(1, OUT_DIM))
    return out


# speedup vs baseline: 6.9888x; 1.9952x over previous
import functools

import jax
import jax.numpy as jnp
from jax import lax
from jax.experimental import pallas as pl
from jax.experimental.pallas import tpu as pltpu
from jax.experimental.pallas import tpu_sc as plsc

N = 10000
E = 160000
IN_DIM = 256
HIDDEN = 512
OUT_DIM = 16

f32 = jnp.float32
i32 = jnp.int32

NCORE = 2
NSUB = 16
CW = 128
NB1 = IN_DIM // CW
NB2 = HIDDEN // CW
CHUNK = 128
EP = E // NSUB
NCH = 80
EPP = NCH * CHUNK

NRANGE = 3
RSPAN = 3456
NPAD = NRANGE * RSPAN
TBL_ROWS = RSPAN + CHUNK
TSTRIPE = TBL_ROWS // NSUB
OR_FULL = RSPAN // NSUB
CAPCH = 88

_mesh = plsc.VectorSubcoreMesh(core_axis_name="c", subcore_axis_name="s")


def _fill(buf, rows, width, value):
    lanes = width // 16

    def body(i, carry):
        r = i // lanes
        col = (i % lanes) * 16
        buf[r, pl.ds(col, 16)] = jnp.full((16,), value, f32)
        return carry

    lax.fori_loop(0, rows * lanes, body, 0)


@functools.partial(
    pl.kernel,
    out_type=[jax.ShapeDtypeStruct((NPAD, 16), f32),
              jax.ShapeDtypeStruct((NSUB, CAPCH, CHUNK), i32),
              jax.ShapeDtypeStruct((NSUB, CAPCH, CHUNK), i32),
              jax.ShapeDtypeStruct((NSUB, 16), i32)],
    mesh=_mesh,
    compiler_params=pltpu.CompilerParams(needs_layout_passes=False),
    scratch_types=[pltpu.VMEM((NCH, CHUNK), i32)] * 2 + [
        pltpu.VMEM((CAPCH, CHUNK), i32),
        pltpu.VMEM((CAPCH, CHUNK), i32),
        pltpu.VMEM((CHUNK, 16), f32),
        pltpu.VMEM((TSTRIPE, 16), f32),
        pltpu.VMEM((16,), i32),
        pltpu.VMEM_SHARED((TBL_ROWS, 16), f32),
    ],
)
def _prep_kernel(src_hbm, dst_hbm, *rest):
    (deg_hbm, bsrc_hbm, bdst_hbm, cnt_hbm,
     src_v, dst_v, bsrc_v, bdst_v, ones_v, init_v, cnt_v, table) = rest
    c = lax.axis_index("c")
    s = lax.axis_index("s")
    pltpu.sync_copy(src_hbm.at[s], src_v)
    pltpu.sync_copy(dst_hbm.at[s], dst_v)
    _fill(ones_v, CHUNK, 16, 1.0)
    _fill(init_v, TSTRIPE, 16, 1.0)

    iota = lax.iota(i32, 16)
    zero16 = jnp.zeros((16,), i32)

    cnt_all = zero16
    start = 0
    starts = []
    for r in range(NRANGE):
        lo = r * RSPAN
        hi = min((r + 1) * RSPAN, N)
        base = start * CHUNK

        def bucket_body(i, off, r=r, lo=lo, hi=hi):
            row = i // 8
            col = (i % 8) * 16
            vd = dst_v[row, pl.ds(col, 16)]
            vs = src_v[row, pl.ds(col, 16)]
            m = (vd >= lo) & (vd < hi)
            pos = off + plsc.cumsum(m.astype(i32)) - 1
            plsc.store_scatter(bdst_v, [pos // CHUNK, pos % CHUNK],
                               vd - lo, mask=m)
            plsc.store_scatter(bsrc_v, [pos // CHUNK, pos % CHUNK],
                               vs, mask=m)
            return off + plsc.all_reduce_population_count(m)

        off = lax.fori_loop(0, NCH * 8, bucket_body, zero16 + base)

        cnt = jnp.sum(jnp.where(iota == 0, off, 0)) - base
        nch2 = ((((cnt + CHUNK - 1) // CHUNK) + 1) // 2) * 2
        limit = base + nch2 * CHUNK

        def pad_body(k, off2, limit=limit):
            pos = off2 + iota + 16 * k
            m = pos < limit
            plsc.store_scatter(bdst_v, [pos // CHUNK, pos % CHUNK],
                               RSPAN + iota, mask=m)
            plsc.store_scatter(bsrc_v, [pos // CHUNK, pos % CHUNK],
                               zero16, mask=m)
            return off2

        lax.fori_loop(0, 16, pad_body, off)
        cnt_all = cnt_all + jnp.where(iota == r, cnt, 0)
        starts.append(start)
        start = start + nch2
    cnt_v[...] = cnt_all

    @pl.when(c == 0)
    def _():
        pltpu.sync_copy(bsrc_v, bsrc_hbm.at[s])
        pltpu.sync_copy(bdst_v, bdst_hbm.at[s])
        pltpu.sync_copy(cnt_v, cnt_hbm.at[s])

    for r in range(NRANGE):
        @pl.when(c == r % NCORE)
        def _(r=r):
            pltpu.sync_copy(
                init_v,
                table.at[pl.ds(pl.multiple_of(s * TSTRIPE, 8), TSTRIPE)])
            plsc.subcore_barrier()
            nch = (jnp.sum(jnp.where(iota == r, cnt_all, 0))
                   + (CHUNK - 1)) // CHUNK
            rb = starts[r]

            def body(j, carry, rb=rb):
                pltpu.sync_copy(ones_v, table.at[bdst_v.at[rb + j]], add=True)
                return carry

            lax.fori_loop(0, nch, body, 0)
            plsc.subcore_barrier()
            pltpu.sync_copy(
                table.at[pl.ds(pl.multiple_of(s * OR_FULL, 8), OR_FULL)],
                deg_hbm.at[pl.ds(pl.multiple_of(r * RSPAN + s * OR_FULL, 8),
                                 OR_FULL)])
            plsc.subcore_barrier()


def _make_prop(nb):
    @functools.partial(
        pl.kernel,
        out_type=[jax.ShapeDtypeStruct((NPAD, CW), f32) for _ in range(nb)],
        mesh=_mesh,
        compiler_params=pltpu.CompilerParams(needs_layout_passes=False),
        scratch_types=[
            pltpu.VMEM((CAPCH, CHUNK), i32),
            pltpu.VMEM((CAPCH, CHUNK), i32),
            pltpu.VMEM((16,), i32),
            pltpu.VMEM((CHUNK, CW), f32),
            pltpu.VMEM((CHUNK, CW), f32),
            pltpu.VMEM((TSTRIPE // 2, CW), f32),
            pltpu.VMEM_SHARED((TBL_ROWS, CW), f32),
            pltpu.SemaphoreType.DMA,
            pltpu.SemaphoreType.DMA,
        ],
    )
    def _prop(*args):
        bsrc_hbm, bdst_hbm, cnt_hbm = args[:3]
        tbls = args[3:3 + nb]
        outs = args[3 + nb:3 + 2 * nb]
        sc = args[3 + 2 * nb:]
        bsrc_v, bdst_v, cnt_v, buf0, buf1, zbuf, table, sem0, sem1 = sc
        c = lax.axis_index("c")
        s = lax.axis_index("s")
        pltpu.sync_copy(bsrc_hbm.at[s], bsrc_v)
        pltpu.sync_copy(bdst_hbm.at[s], bdst_v)
        pltpu.sync_copy(cnt_hbm.at[s], cnt_v)
        cnt_all = cnt_v[...]
        iota = lax.iota(i32, 16)
        nch2_vec = jnp.zeros((16,), i32)
        start_vec = jnp.zeros((16,), i32)
        srt = 0
        for r in range(NRANGE):
            cnt = jnp.sum(jnp.where(iota == r, cnt_all, 0))
            nch2 = ((((cnt + CHUNK - 1) // CHUNK) + 1) // 2) * 2
            nch2_vec = nch2_vec + jnp.where(iota == r, nch2, 0)
            start_vec = start_vec + jnp.where(iota == r, srt, 0)
            srt = srt + nch2
        half = TSTRIPE // 2
        _fill(zbuf, half, CW, 0.0)

        def make_phase(tbl, out):
            def phase(r, carry):
                pltpu.sync_copy(
                    zbuf,
                    table.at[pl.ds(pl.multiple_of(s * TSTRIPE, 8), half)])
                pltpu.sync_copy(
                    zbuf,
                    table.at[pl.ds(pl.multiple_of(s * TSTRIPE + half, 8),
                                   half)])
                plsc.subcore_barrier()
                nch2 = jnp.sum(jnp.where(iota == r, nch2_vec, 0))
                rb = jnp.sum(jnp.where(iota == r, start_vec, 0))

                @pl.when(nch2 > 0)
                def _():
                    pltpu.async_copy(tbl.at[bsrc_v.at[rb]], buf0, sem0)

                def body(i, carry2):
                    j0 = rb + 2 * i
                    pltpu.make_async_copy(tbl.at[bsrc_v.at[j0]],
                                          buf0, sem0).wait()
                    pltpu.async_copy(tbl.at[bsrc_v.at[j0 + 1]], buf1, sem1)
                    pltpu.sync_copy(buf0, table.at[bdst_v.at[j0]], add=True)
                    pltpu.make_async_copy(tbl.at[bsrc_v.at[j0 + 1]],
                                          buf1, sem1).wait()

                    @pl.when(2 * i + 2 < nch2)
                    def _():
                        pltpu.async_copy(tbl.at[bsrc_v.at[j0 + 2]],
                                         buf0, sem0)

                    pltpu.sync_copy(buf1, table.at[bdst_v.at[j0 + 1]],
                                    add=True)
                    return carry2

                lax.fori_loop(0, nch2 // 2, body, 0)
                plsc.subcore_barrier()
                pltpu.sync_copy(
                    table.at[pl.ds(pl.multiple_of(s * OR_FULL, 8), OR_FULL)],
                    out.at[pl.ds(pl.multiple_of(r * RSPAN + s * OR_FULL, 8),
                                 OR_FULL)])
                plsc.subcore_barrier()
                return carry

            return phase

        for cc in range(NCORE):
            @pl.when(c == cc)
            def _(cc=cc):
                for cb in range(cc, nb, NCORE):
                    lax.fori_loop(0, NRANGE,
                                  make_phase(tbls[cb], outs[cb]), 0)

    return _prop


_prop_l1 = _make_prop(NB1)
_prop_l2 = _make_prop(NB2)


RB = 1000
GRID = N // RB


def _row_spec(width):
    return pl.BlockSpec((RB, width), lambda i: (i, 0))


def _full_spec(r, c):
    return pl.BlockSpec((r, c), lambda i: (0, 0))


def _tc_a_body(deg_ref, x_ref, dis_ref, *outs):
    d = lax.rsqrt(deg_ref[...])
    dis_ref[...] = d
    xs = x_ref[...] * d[:, 0:1]
    for k in range(NB1):
        outs[k][...] = xs[:, CW * k:CW * (k + 1)]


_tc_a = pl.pallas_call(
    _tc_a_body,
    grid=(GRID,),
    in_specs=[_row_spec(16), _row_spec(IN_DIM)],
    out_specs=[_row_spec(16)] + [_row_spec(CW)] * NB1,
    out_shape=[jax.ShapeDtypeStruct((N, 16), f32)]
    + [jax.ShapeDtypeStruct((N, CW), f32) for _ in range(NB1)],
)


def _tc_b_body(*refs):
    gs = refs[:NB1]
    x_ref, dis_ref, w1_ref, b1_ref = refs[NB1:NB1 + 4]
    outs = refs[NB1 + 4:]
    d = dis_ref[:, 0:1]
    g = jnp.concatenate([r[...] for r in gs], axis=1)
    p = d * (g + d * x_ref[...])
    h = jnp.dot(p, w1_ref[...], preferred_element_type=f32) + b1_ref[...]
    hs = d * jnp.maximum(h, 0.0)
    for k in range(NB2):
        outs[k][...] = hs[:, CW * k:CW * (k + 1)]


_tc_b = pl.pallas_call(
    _tc_b_body,
    grid=(GRID,),
    in_specs=[_row_spec(CW)] * NB1 + [_row_spec(IN_DIM), _row_spec(16),
              _full_spec(IN_DIM, HIDDEN), _full_spec(1, HIDDEN)],
    out_specs=[_row_spec(CW)] * NB2,
    out_shape=[jax.ShapeDtypeStruct((N, CW), f32) for _ in range(NB2)],
)


def _tc_c_body(*refs):
    gs = refs[:NB2]
    hs_refs = refs[NB2:2 * NB2]
    dis_ref, w2_ref, b2_ref, wc_ref, bc_ref, out_ref = refs[2 * NB2:]
    d = dis_ref[:, 0:1]
    g = jnp.concatenate([r[...] for r in gs], axis=1)
    hs = jnp.concatenate([r[...] for r in hs_refs], axis=1)
    p = d * (g + hs)
    h = jnp.dot(p, w2_ref[...], preferred_element_type=f32) + b2_ref[...]
    h = jnp.maximum(h, 0.0)
    out_ref[...] = jnp.dot(h, wc_ref[...], preferred_element_type=f32) + bc_ref[...]


_tc_c = pl.pallas_call(
    _tc_c_body,
    grid=(GRID,),
    in_specs=[_row_spec(CW)] * (2 * NB2) + [_row_spec(16),
              _full_spec(HIDDEN, HIDDEN), _full_spec(1, HIDDEN),
              _full_spec(HIDDEN, OUT_DIM), _full_spec(1, OUT_DIM)],
    out_specs=_row_spec(OUT_DIM),
    out_shape=jax.ShapeDtypeStruct((N, OUT_DIM), f32),
)


def kernel(x, edge_index, W1, b1, W2, b2, Wc, bc):
    src = edge_index[0].astype(i32)
    dst = edge_index[1].astype(i32)
    pad = EPP - EP
    srcr = jnp.concatenate(
        [src.reshape(NSUB, EP), jnp.zeros((NSUB, pad), i32)],
        axis=1).reshape(NSUB, NCH, CHUNK)
    dstr = jnp.concatenate(
        [dst.reshape(NSUB, EP), jnp.full((NSUB, pad), N, i32)],
        axis=1).reshape(NSUB, NCH, CHUNK)

    deg16, bsrc, bdst, cnts = _prep_kernel(srcr, dstr)
    dis, *xs = _tc_a(deg16, x)
    g1 = _prop_l1(bsrc, bdst, cnts, *xs)
    hs = _tc_b(*g1, x, dis, W1, b1.reshape(1, HIDDEN))
    g2 = _prop_l2(bsrc, bdst, cnts, *hs)
    out = _tc_c(*g2, *hs, dis, W2, b2.reshape(1, HIDDEN),
                Wc, bc.reshape(1, OUT_DIM))
    return out
